# baseline (speedup 1.0000x reference)
import functools

import jax
import jax.numpy as jnp
from jax import lax
from jax.experimental import pallas as pl
from jax.experimental.pallas import tpu as pltpu
from jax.experimental.pallas import tpu_sc as plsc

_B = 16384
_D = 64
_LANES = 16


def _score_body(nodes_hbm, rel_hbm, s_hbm, o_hbm, r_hbm, out_hbm,
                sidx_v, oidx_v, ridx_v, srows_v, orows_v, rrows_v,
                out_v, sem):
    info = plsc.get_sparse_core_info()
    nw = info.num_cores * info.num_subcores
    bpw = _B // nw
    wid = lax.axis_index("s") * info.num_cores + lax.axis_index("c")
    base = wid * bpw

    pltpu.sync_copy(s_hbm.at[pl.ds(base, bpw)], sidx_v)
    pltpu.sync_copy(o_hbm.at[pl.ds(base, bpw)], oidx_v)
    pltpu.sync_copy(r_hbm.at[pl.ds(base, bpw)], ridx_v)

    cs = pltpu.async_copy(nodes_hbm.at[sidx_v], srows_v, sem)
    co = pltpu.async_copy(nodes_hbm.at[oidx_v], orows_v, sem)
    cr = pltpu.async_copy(rel_hbm.at[ridx_v], rrows_v, sem)
    cs.wait()
    co.wait()
    cr.wait()

    lanes = lax.iota(jnp.int32, _LANES)

    def group(g, carry):
        row0 = g * _LANES
        tot = jnp.zeros((_LANES,), jnp.float32)
        for i in range(_LANES):
            sl = pl.ds(0, _LANES)
            acc = (srows_v[row0 + i, sl] * rrows_v[row0 + i, sl]
                   * orows_v[row0 + i, sl])
            for j in range(1, _D // _LANES):
                sl = pl.ds(j * _LANES, _LANES)
                acc = acc + (srows_v[row0 + i, sl] * rrows_v[row0 + i, sl]
                             * orows_v[row0 + i, sl])
            tot = jnp.where(lanes == i, jnp.sum(acc), tot)
        out_v[pl.ds(row0, _LANES)] = tot
        return carry

    lax.fori_loop(0, bpw // _LANES, group, 0)

    pltpu.sync_copy(out_v, out_hbm.at[pl.ds(base, bpw)])


def kernel(node_embeddings, s, o, r, rel_embedding):
    info = plsc.get_sparse_core_info()
    nw = info.num_cores * info.num_subcores
    bpw = _B // nw
    mesh = plsc.VectorSubcoreMesh(core_axis_name="c", subcore_axis_name="s")
    run = pl.kernel(
        _score_body,
        out_type=jax.ShapeDtypeStruct((_B,), jnp.float32),
        mesh=mesh,
        compiler_params=pltpu.CompilerParams(needs_layout_passes=False,
                                             use_tc_tiling_on_sc=False),
        scratch_types=[
            pltpu.VMEM((bpw,), jnp.int32),
            pltpu.VMEM((bpw,), jnp.int32),
            pltpu.VMEM((bpw,), jnp.int32),
            pltpu.VMEM((bpw, _D), jnp.float32),
            pltpu.VMEM((bpw, _D), jnp.float32),
            pltpu.VMEM((bpw, _D), jnp.float32),
            pltpu.VMEM((bpw,), jnp.float32),
            pltpu.SemaphoreType.DMA,
        ],
    )
    return run(node_embeddings, rel_embedding,
               s.astype(jnp.int32), o.astype(jnp.int32), r.astype(jnp.int32))
